# baseline (speedup 1.0000x reference)
import functools

import jax
import jax.numpy as jnp
from jax import lax
from jax.experimental import pallas as pl
from jax.experimental.pallas import tpu as pltpu
from jax.experimental.pallas import tpu_sc as plsc

_N_TERMS = 104
_A_PAD = 112


def _tc_forward(ct, a_row, b_off, b_count):
    C, B, T = ct.shape
    BBLK = 256

    def body(ct_ref, a_ref, out_ref):
        idx = lax.broadcasted_iota(jnp.int32, (8, _A_PAD), 1)
        av = jnp.broadcast_to(a_ref[...], (8, _A_PAD))
        us = (idx % 3) == 0
        z = ((us & (av > 0.0)) | (~us & (av < 0.0))) & (idx >= 2)
        ae = jnp.where(z, 0.0, av)

        def AE(i):
            return ae[0, i]

        d = [ct_ref[22 + m] for m in range(3)]
        acc = ct_ref[0] * AE(0)
        vcur = ct_ref[1]
        for c in range(1, 22):
            x = vcur
            if c <= 20:
                vn = ct_ref[c + 1]
            t1 = AE(c) + AE(21 + c) * d[(c - 1) % 3]
            t2 = AE(82 + c)
            if c <= 20:
                t1 = t1 + AE(42 + c) * vn
                t2 = t2 + AE(62 + c) * vn
            r = 1.0 / (x + 0.5)
            acc = acc + x * (t1 + r * t2)
            if c <= 20:
                vcur = vn
        out_ref[...] = acc

    blk0 = b_off // BBLK
    return pl.pallas_call(
        body,
        grid=(b_count // BBLK,),
        in_specs=[
            pl.BlockSpec((C, BBLK, T), lambda i: (0, i + blk0, 0)),
            pl.BlockSpec((1, _A_PAD), lambda i: (0, 0)),
        ],
        out_specs=pl.BlockSpec((BBLK, T), lambda i: (i, 0)),
        out_shape=jax.ShapeDtypeStruct((b_count, T), jnp.float32),
    )(ct, a_row)


def _sc_forward(ct, a_pad, b_sc):
    C, B, T = ct.shape
    info = plsc.get_sparse_core_info()
    NC, NS = info.num_cores, info.num_subcores
    NW = NC * NS
    b_per_w = b_sc // NW
    n_groups = (T + 15) // 16
    t_pad = 16 * n_groups
    GPS = 4
    n_stripes = n_groups // GPS
    CH_B = 8
    n_chunks = b_per_w // CH_B

    mesh = plsc.VectorSubcoreMesh(core_axis_name="c", subcore_axis_name="s")

    @functools.partial(
        pl.kernel,
        mesh=mesh,
        out_type=jax.ShapeDtypeStruct((b_sc * T,), jnp.float32),
        compiler_params=pltpu.CompilerParams(needs_layout_passes=False),
        scratch_types=[
            pltpu.VMEM((_A_PAD,), jnp.float32),
            pltpu.VMEM((_A_PAD,), jnp.float32),
            pltpu.VMEM((2 * C, CH_B, T), jnp.float32),
            pltpu.VMEM((CH_B * T + 24,), jnp.float32),
            pltpu.SemaphoreType.DMA,
            pltpu.SemaphoreType.DMA,
        ],
    )
    def run(ct_hbm, a_hbm, out_hbm, a_v, ae_v, buf, obuf, sem0, sem1):
        wid = lax.axis_index("s") * NC + lax.axis_index("c")
        first = wid * b_per_w
        sems = (sem0, sem1)

        pltpu.sync_copy(a_hbm, a_v)
        for k in range(_A_PAD // 16):
            idx = lax.iota(jnp.int32, 16) + 16 * k
            av = a_v[pl.ds(16 * k, 16)]
            us = (idx % 3) == 0
            z = jnp.where(us, av > 0.0, av < 0.0) & (idx >= 2)
            ae_v[pl.ds(16 * k, 16)] = jnp.where(z, 0.0, av)

        ae_regs = [ae_v[pl.ds(16 * k, 16)] for k in range(_A_PAD // 16)]

        def AE(i):
            return ae_regs[i // 16][i % 16]

        riota = lax.iota(jnp.int32, 16)

        def fire_chunk(ci, slot):
            b0 = first + ci * CH_B
            return [
                pltpu.async_copy(
                    ct_hbm.at[c, pl.ds(b0, CH_B)],
                    buf.at[slot * C + c],
                    sems[slot],
                )
                for c in range(C)
            ]

        def wait_chunk(ci, slot):
            b0 = first + ci * CH_B
            for c in range(C):
                pltpu.make_async_copy(
                    ct_hbm.at[c, pl.ds(b0, CH_B)],
                    buf.at[slot * C + c],
                    sems[slot],
                ).wait()

        def compute_chunk(slot):
            def traj(b, carry):
                def stripe(s, carry2):
                    off = 64 * s

                    def ld(c, k):
                        return buf[slot * C + c, b, pl.ds(off + 16 * k, 16)]

                    d = [[ld(22 + m, k) for k in range(GPS)] for m in range(3)]
                    acc = [ld(0, k) * AE(0) for k in range(GPS)]
                    vcur = [ld(1, k) for k in range(GPS)]
                    for c in range(1, 22):
                        if c <= 20:
                            vn = [ld(c + 1, k) for k in range(GPS)]
                        cl, cd = AE(c), AE(21 + c)
                        ch = AE(82 + c)
                        if c <= 20:
                            cb, cm = AE(42 + c), AE(62 + c)
                        dm = d[(c - 1) % 3]
                        for k in range(GPS):
                            x = vcur[k]
                            t1 = cl + cd * dm[k]
                            t2 = ch
                            if c <= 20:
                                t1 = t1 + cb * vn[k]
                                t2 = t2 + cm * vn[k]
                            r = 1.0 / (x + 0.5)
                            acc[k] = acc[k] + x * (t1 + r * t2)
                        if c <= 20:
                            vcur = vn
                    for k in range(GPS):
                        p = riota + (off + 16 * k)
                        plsc.store_scatter(
                            obuf, [p + T * b], acc[k], mask=p < T
                        )
                    return carry2

                lax.fori_loop(0, n_stripes, stripe, 0)
                return carry

            lax.fori_loop(0, CH_B, traj, 0)

        fire_chunk(0, 0)

        def body(i, carry):
            for j in range(2):
                ci = 2 * i + j
                nci = jnp.minimum(ci + 1, n_chunks - 1)
                fire_chunk(nci, 1 - j)
                wait_chunk(ci, j)
                compute_chunk(j)
                base = (first + ci * CH_B) * T
                pltpu.sync_copy(
                    obuf.at[pl.ds(0, CH_B * T)],
                    out_hbm.at[pl.ds(base, CH_B * T)],
                )
            return carry

        lax.fori_loop(0, n_chunks // 2, body, 0)
        wait_chunk(n_chunks - 1, 0)

    return run(ct, a_pad)


_B_SC = 512


def kernel(candidates, a):
    B, T, _ = candidates.shape
    ct = jnp.transpose(candidates, (2, 0, 1))
    a_pad = jnp.zeros((_A_PAD,), jnp.float32).at[:_N_TERMS].set(a)
    sc_out = _sc_forward(ct, a_pad, _B_SC)
    tc_out = _tc_forward(ct, a_pad.reshape(1, _A_PAD), _B_SC, B - _B_SC)
    return jnp.concatenate([sc_out.reshape(_B_SC, T), tc_out], axis=0)


# speedup vs baseline: 1.0858x; 1.0858x over previous
import functools

import jax
import jax.numpy as jnp
from jax import lax
from jax.experimental import pallas as pl
from jax.experimental.pallas import tpu as pltpu
from jax.experimental.pallas import tpu_sc as plsc

_N_TERMS = 104
_A_PAD = 112


def _tc_forward(ct, a_row, b_off, b_count):
    C, B, T = ct.shape
    BBLK = 256

    def body(ct_ref, a_ref, out_ref):
        idx = lax.broadcasted_iota(jnp.int32, (8, _A_PAD), 1)
        av = jnp.broadcast_to(a_ref[...], (8, _A_PAD))
        us = (idx % 3) == 0
        z = ((us & (av > 0.0)) | (~us & (av < 0.0))) & (idx >= 2)
        ae = jnp.where(z, 0.0, av)

        def AE(i):
            return ae[0, i]

        d = [ct_ref[22 + m] for m in range(3)]
        acc = ct_ref[0] * AE(0)
        vcur = ct_ref[1]
        for c in range(1, 22):
            x = vcur
            if c <= 20:
                vn = ct_ref[c + 1]
            t1 = AE(c) + AE(21 + c) * d[(c - 1) % 3]
            t2 = AE(82 + c)
            if c <= 20:
                t1 = t1 + AE(42 + c) * vn
                t2 = t2 + AE(62 + c) * vn
            r = 1.0 / (x + 0.5)
            acc = acc + x * (t1 + r * t2)
            if c <= 20:
                vcur = vn
        out_ref[...] = acc

    blk0 = b_off // BBLK
    return pl.pallas_call(
        body,
        grid=(b_count // BBLK,),
        in_specs=[
            pl.BlockSpec((C, BBLK, T), lambda i: (0, i + blk0, 0)),
            pl.BlockSpec((1, _A_PAD), lambda i: (0, 0)),
        ],
        out_specs=pl.BlockSpec((BBLK, T), lambda i: (i, 0)),
        out_shape=jax.ShapeDtypeStruct((b_count, T), jnp.float32),
    )(ct, a_row)


def _sc_forward(ct, a_pad, b_sc):
    C, B, T = ct.shape
    info = plsc.get_sparse_core_info()
    NC, NS = info.num_cores, info.num_subcores
    NW = NC * NS
    b_per_w = b_sc // NW
    n_groups = (T + 15) // 16
    t_pad = 16 * n_groups
    GPS = 4
    n_stripes = n_groups // GPS
    CH_B = 8
    n_chunks = b_per_w // CH_B

    mesh = plsc.VectorSubcoreMesh(core_axis_name="c", subcore_axis_name="s")

    @functools.partial(
        pl.kernel,
        mesh=mesh,
        out_type=jax.ShapeDtypeStruct((b_sc * T,), jnp.float32),
        compiler_params=pltpu.CompilerParams(needs_layout_passes=False),
        scratch_types=[
            pltpu.VMEM((_A_PAD,), jnp.float32),
            pltpu.VMEM((_A_PAD,), jnp.float32),
            pltpu.VMEM((2 * C, CH_B, T), jnp.float32),
            pltpu.VMEM((CH_B * T + 24,), jnp.float32),
            pltpu.SemaphoreType.DMA,
            pltpu.SemaphoreType.DMA,
        ],
    )
    def run(ct_hbm, a_hbm, out_hbm, a_v, ae_v, buf, obuf, sem0, sem1):
        wid = lax.axis_index("s") * NC + lax.axis_index("c")
        first = wid * b_per_w
        sems = (sem0, sem1)

        pltpu.sync_copy(a_hbm, a_v)
        for k in range(_A_PAD // 16):
            idx = lax.iota(jnp.int32, 16) + 16 * k
            av = a_v[pl.ds(16 * k, 16)]
            us = (idx % 3) == 0
            z = jnp.where(us, av > 0.0, av < 0.0) & (idx >= 2)
            ae_v[pl.ds(16 * k, 16)] = jnp.where(z, 0.0, av)

        ae_regs = [ae_v[pl.ds(16 * k, 16)] for k in range(_A_PAD // 16)]

        def AE(i):
            return ae_regs[i // 16][i % 16]

        riota = lax.iota(jnp.int32, 16)

        def fire_chunk(ci, slot):
            b0 = first + ci * CH_B
            return [
                pltpu.async_copy(
                    ct_hbm.at[c, pl.ds(b0, CH_B)],
                    buf.at[slot * C + c],
                    sems[slot],
                )
                for c in range(C)
            ]

        def wait_chunk(ci, slot):
            b0 = first + ci * CH_B
            for c in range(C):
                pltpu.make_async_copy(
                    ct_hbm.at[c, pl.ds(b0, CH_B)],
                    buf.at[slot * C + c],
                    sems[slot],
                ).wait()

        def compute_chunk(slot):
            def traj(b, carry):
                def stripe(s, carry2):
                    off = 64 * s

                    def ld(c, k):
                        return buf[slot * C + c, b, pl.ds(off + 16 * k, 16)]

                    d = [[ld(22 + m, k) for k in range(GPS)] for m in range(3)]
                    acc = [ld(0, k) * AE(0) for k in range(GPS)]
                    vcur = [ld(1, k) for k in range(GPS)]
                    for c in range(1, 22):
                        if c <= 20:
                            vn = [ld(c + 1, k) for k in range(GPS)]
                        cl, cd = AE(c), AE(21 + c)
                        ch = AE(82 + c)
                        if c <= 20:
                            cb, cm = AE(42 + c), AE(62 + c)
                        dm = d[(c - 1) % 3]
                        for k in range(GPS):
                            x = vcur[k]
                            t1 = cl + cd * dm[k]
                            t2 = ch
                            if c <= 20:
                                t1 = t1 + cb * vn[k]
                                t2 = t2 + cm * vn[k]
                            r = 1.0 / (x + 0.5)
                            acc[k] = acc[k] + x * (t1 + r * t2)
                        if c <= 20:
                            vcur = vn
                    for k in range(GPS):
                        p = riota + (off + 16 * k)
                        plsc.store_scatter(
                            obuf, [p + T * b], acc[k], mask=p < T
                        )
                    return carry2

                lax.fori_loop(0, n_stripes, stripe, 0)
                return carry

            lax.fori_loop(0, CH_B, traj, 0)

        fire_chunk(0, 0)

        def body(i, carry):
            for j in range(2):
                ci = 2 * i + j
                nci = jnp.minimum(ci + 1, n_chunks - 1)
                fire_chunk(nci, 1 - j)
                wait_chunk(ci, j)
                compute_chunk(j)
                base = (first + ci * CH_B) * T
                pltpu.sync_copy(
                    obuf.at[pl.ds(0, CH_B * T)],
                    out_hbm.at[pl.ds(base, CH_B * T)],
                )
            return carry

        lax.fori_loop(0, n_chunks // 2, body, 0)
        wait_chunk(n_chunks - 1, 0)

    return run(ct, a_pad)


_B_SC = 1024


def kernel(candidates, a):
    B, T, _ = candidates.shape
    ct = jnp.transpose(candidates, (2, 0, 1))
    a_pad = jnp.zeros((_A_PAD,), jnp.float32).at[:_N_TERMS].set(a)
    sc_out = _sc_forward(ct, a_pad, _B_SC)
    tc_out = _tc_forward(ct, a_pad.reshape(1, _A_PAD), _B_SC, B - _B_SC)
    return jnp.concatenate([sc_out.reshape(_B_SC, T), tc_out], axis=0)


# speedup vs baseline: 1.1242x; 1.0353x over previous
import functools

import jax
import jax.numpy as jnp
from jax import lax
from jax.experimental import pallas as pl
from jax.experimental.pallas import tpu as pltpu
from jax.experimental.pallas import tpu_sc as plsc

_N_TERMS = 104
_A_PAD = 112


def _tc_forward(ct, a_row, b_off, b_count):
    C, B, T = ct.shape
    BBLK = 256

    def body(ct_ref, a_ref, out_ref):
        idx = lax.broadcasted_iota(jnp.int32, (8, _A_PAD), 1)
        av = jnp.broadcast_to(a_ref[...], (8, _A_PAD))
        us = (idx % 3) == 0
        z = ((us & (av > 0.0)) | (~us & (av < 0.0))) & (idx >= 2)
        ae = jnp.where(z, 0.0, av)

        def AE(i):
            return ae[0, i]

        d = [ct_ref[22 + m] for m in range(3)]
        acc = ct_ref[0] * AE(0)
        vcur = ct_ref[1]
        for c in range(1, 22):
            x = vcur
            if c <= 20:
                vn = ct_ref[c + 1]
            t1 = AE(c) + AE(21 + c) * d[(c - 1) % 3]
            t2 = AE(82 + c)
            if c <= 20:
                t1 = t1 + AE(42 + c) * vn
                t2 = t2 + AE(62 + c) * vn
            r = 1.0 / (x + 0.5)
            acc = acc + x * (t1 + r * t2)
            if c <= 20:
                vcur = vn
        out_ref[...] = acc

    blk0 = b_off // BBLK
    return pl.pallas_call(
        body,
        grid=(b_count // BBLK,),
        in_specs=[
            pl.BlockSpec((C, BBLK, T), lambda i: (0, i + blk0, 0)),
            pl.BlockSpec((1, _A_PAD), lambda i: (0, 0)),
        ],
        out_specs=pl.BlockSpec((BBLK, T), lambda i: (i, 0)),
        out_shape=jax.ShapeDtypeStruct((b_count, T), jnp.float32),
    )(ct, a_row)


def _sc_forward(ct, a_pad, b_sc):
    C, B, T = ct.shape
    info = plsc.get_sparse_core_info()
    NC, NS = info.num_cores, info.num_subcores
    NW = NC * NS
    b_per_w = b_sc // NW
    n_groups = (T + 15) // 16
    t_pad = 16 * n_groups
    GPS = 4
    n_stripes = n_groups // GPS
    CH_B = 8
    n_chunks = b_per_w // CH_B

    mesh = plsc.VectorSubcoreMesh(core_axis_name="c", subcore_axis_name="s")

    @functools.partial(
        pl.kernel,
        mesh=mesh,
        out_type=jax.ShapeDtypeStruct((b_sc, t_pad), jnp.float32),
        compiler_params=pltpu.CompilerParams(needs_layout_passes=False),
        scratch_types=[
            pltpu.VMEM((_A_PAD,), jnp.float32),
            pltpu.VMEM((_A_PAD,), jnp.float32),
            pltpu.VMEM((2 * C, CH_B, T), jnp.float32),
            pltpu.VMEM((CH_B, t_pad), jnp.float32),
            pltpu.SemaphoreType.DMA,
            pltpu.SemaphoreType.DMA,
        ],
    )
    def run(ct_hbm, a_hbm, out_hbm, a_v, ae_v, buf, obuf, sem0, sem1):
        wid = lax.axis_index("s") * NC + lax.axis_index("c")
        first = wid * b_per_w
        sems = (sem0, sem1)

        pltpu.sync_copy(a_hbm, a_v)
        for k in range(_A_PAD // 16):
            idx = lax.iota(jnp.int32, 16) + 16 * k
            av = a_v[pl.ds(16 * k, 16)]
            us = (idx % 3) == 0
            z = jnp.where(us, av > 0.0, av < 0.0) & (idx >= 2)
            ae_v[pl.ds(16 * k, 16)] = jnp.where(z, 0.0, av)

        ae_regs = [ae_v[pl.ds(16 * k, 16)] for k in range(_A_PAD // 16)]

        def AE(i):
            return ae_regs[i // 16][i % 16]

        riota = lax.iota(jnp.int32, 16)

        def fire_chunk(ci, slot):
            b0 = first + ci * CH_B
            return [
                pltpu.async_copy(
                    ct_hbm.at[c, pl.ds(b0, CH_B)],
                    buf.at[slot * C + c],
                    sems[slot],
                )
                for c in range(C)
            ]

        def wait_chunk(ci, slot):
            b0 = first + ci * CH_B
            for c in range(C):
                pltpu.make_async_copy(
                    ct_hbm.at[c, pl.ds(b0, CH_B)],
                    buf.at[slot * C + c],
                    sems[slot],
                ).wait()

        def compute_chunk(slot):
            def traj(b, carry):
                def stripe(s, carry2):
                    off = 64 * s

                    def ld(c, k):
                        return buf[slot * C + c, b, pl.ds(off + 16 * k, 16)]

                    d = [[ld(22 + m, k) for k in range(GPS)] for m in range(3)]
                    acc = [ld(0, k) * AE(0) for k in range(GPS)]
                    vcur = [ld(1, k) for k in range(GPS)]
                    for c in range(1, 22):
                        if c <= 20:
                            vn = [ld(c + 1, k) for k in range(GPS)]
                        cl, cd = AE(c), AE(21 + c)
                        ch = AE(82 + c)
                        if c <= 20:
                            cb, cm = AE(42 + c), AE(62 + c)
                        dm = d[(c - 1) % 3]
                        for k in range(GPS):
                            x = vcur[k]
                            t1 = cl + cd * dm[k]
                            t2 = ch
                            if c <= 20:
                                t1 = t1 + cb * vn[k]
                                t2 = t2 + cm * vn[k]
                            r = 1.0 / (x + 0.5)
                            acc[k] = acc[k] + x * (t1 + r * t2)
                        if c <= 20:
                            vcur = vn
                    for k in range(GPS):
                        obuf[b, pl.ds(off + 16 * k, 16)] = acc[k]
                    return carry2

                lax.fori_loop(0, n_stripes, stripe, 0)
                return carry

            lax.fori_loop(0, CH_B, traj, 0)

        fire_chunk(0, 0)

        def body(i, carry):
            for j in range(2):
                ci = 2 * i + j
                nci = jnp.minimum(ci + 1, n_chunks - 1)
                fire_chunk(nci, 1 - j)
                wait_chunk(ci, j)
                compute_chunk(j)
                pltpu.sync_copy(
                    obuf, out_hbm.at[pl.ds(first + ci * CH_B, CH_B)]
                )
            return carry

        lax.fori_loop(0, n_chunks // 2, body, 0)
        wait_chunk(n_chunks - 1, 0)

    return run(ct, a_pad)


_B_SC = 1024


def kernel(candidates, a):
    B, T, _ = candidates.shape
    ct = jnp.transpose(candidates, (2, 0, 1))
    a_pad = jnp.zeros((_A_PAD,), jnp.float32).at[:_N_TERMS].set(a)
    sc_out = _sc_forward(ct, a_pad, _B_SC)
    tc_out = _tc_forward(ct, a_pad.reshape(1, _A_PAD), _B_SC, B - _B_SC)
    return jnp.concatenate([sc_out[:, :T], tc_out], axis=0)


# speedup vs baseline: 1.1351x; 1.0097x over previous
import functools

import jax
import jax.numpy as jnp
from jax import lax
from jax.experimental import pallas as pl
from jax.experimental.pallas import tpu as pltpu
from jax.experimental.pallas import tpu_sc as plsc

_N_TERMS = 104
_A_PAD = 112


def _tc_forward(ct, a_row, b_off, b_count):
    C, B, T = ct.shape
    BBLK = 128

    def body(ct_ref, a_ref, out_ref):
        idx = lax.broadcasted_iota(jnp.int32, (8, _A_PAD), 1)
        av = jnp.broadcast_to(a_ref[...], (8, _A_PAD))
        us = (idx % 3) == 0
        z = ((us & (av > 0.0)) | (~us & (av < 0.0))) & (idx >= 2)
        ae = jnp.where(z, 0.0, av)

        def AE(i):
            return ae[0, i]

        d = [ct_ref[22 + m] for m in range(3)]
        acc = ct_ref[0] * AE(0)
        vcur = ct_ref[1]
        for c in range(1, 22):
            x = vcur
            if c <= 20:
                vn = ct_ref[c + 1]
            t1 = AE(c) + AE(21 + c) * d[(c - 1) % 3]
            t2 = AE(82 + c)
            if c <= 20:
                t1 = t1 + AE(42 + c) * vn
                t2 = t2 + AE(62 + c) * vn
            r = 1.0 / (x + 0.5)
            acc = acc + x * (t1 + r * t2)
            if c <= 20:
                vcur = vn
        out_ref[...] = acc

    blk0 = b_off // BBLK
    return pl.pallas_call(
        body,
        grid=(b_count // BBLK,),
        in_specs=[
            pl.BlockSpec((C, BBLK, T), lambda i: (0, i + blk0, 0)),
            pl.BlockSpec((1, _A_PAD), lambda i: (0, 0)),
        ],
        out_specs=pl.BlockSpec((BBLK, T), lambda i: (i, 0)),
        out_shape=jax.ShapeDtypeStruct((b_count, T), jnp.float32),
    )(ct, a_row)


def _sc_forward(ct, a_pad, b_sc):
    C, B, T = ct.shape
    info = plsc.get_sparse_core_info()
    NC, NS = info.num_cores, info.num_subcores
    NW = NC * NS
    b_per_w = b_sc // NW
    n_groups = (T + 15) // 16
    t_pad = 16 * n_groups
    GPS = 4
    n_stripes = n_groups // GPS
    CH_B = 8
    n_chunks = b_per_w // CH_B

    mesh = plsc.VectorSubcoreMesh(core_axis_name="c", subcore_axis_name="s")

    @functools.partial(
        pl.kernel,
        mesh=mesh,
        out_type=jax.ShapeDtypeStruct((b_sc, t_pad), jnp.float32),
        compiler_params=pltpu.CompilerParams(needs_layout_passes=False),
        scratch_types=[
            pltpu.VMEM((_A_PAD,), jnp.float32),
            pltpu.VMEM((_A_PAD,), jnp.float32),
            pltpu.VMEM((2 * C, CH_B, T), jnp.float32),
            pltpu.VMEM((CH_B, t_pad), jnp.float32),
            pltpu.SemaphoreType.DMA,
            pltpu.SemaphoreType.DMA,
        ],
    )
    def run(ct_hbm, a_hbm, out_hbm, a_v, ae_v, buf, obuf, sem0, sem1):
        wid = lax.axis_index("s") * NC + lax.axis_index("c")
        first = wid * b_per_w
        sems = (sem0, sem1)

        pltpu.sync_copy(a_hbm, a_v)
        for k in range(_A_PAD // 16):
            idx = lax.iota(jnp.int32, 16) + 16 * k
            av = a_v[pl.ds(16 * k, 16)]
            us = (idx % 3) == 0
            z = jnp.where(us, av > 0.0, av < 0.0) & (idx >= 2)
            ae_v[pl.ds(16 * k, 16)] = jnp.where(z, 0.0, av)

        ae_regs = [ae_v[pl.ds(16 * k, 16)] for k in range(_A_PAD // 16)]

        def AE(i):
            return ae_regs[i // 16][i % 16]

        riota = lax.iota(jnp.int32, 16)

        def fire_chunk(ci, slot):
            b0 = first + ci * CH_B
            return [
                pltpu.async_copy(
                    ct_hbm.at[c, pl.ds(b0, CH_B)],
                    buf.at[slot * C + c],
                    sems[slot],
                )
                for c in range(C)
            ]

        def wait_chunk(ci, slot):
            b0 = first + ci * CH_B
            for c in range(C):
                pltpu.make_async_copy(
                    ct_hbm.at[c, pl.ds(b0, CH_B)],
                    buf.at[slot * C + c],
                    sems[slot],
                ).wait()

        def compute_chunk(slot):
            def traj(b, carry):
                def stripe(s, carry2):
                    off = 64 * s

                    def ld(c, k):
                        return buf[slot * C + c, b, pl.ds(off + 16 * k, 16)]

                    d = [[ld(22 + m, k) for k in range(GPS)] for m in range(3)]
                    acc = [ld(0, k) * AE(0) for k in range(GPS)]
                    vcur = [ld(1, k) for k in range(GPS)]
                    for c in range(1, 22):
                        if c <= 20:
                            vn = [ld(c + 1, k) for k in range(GPS)]
                        cl, cd = AE(c), AE(21 + c)
                        ch = AE(82 + c)
                        if c <= 20:
                            cb, cm = AE(42 + c), AE(62 + c)
                        dm = d[(c - 1) % 3]
                        for k in range(GPS):
                            x = vcur[k]
                            t1 = cl + cd * dm[k]
                            t2 = ch
                            if c <= 20:
                                t1 = t1 + cb * vn[k]
                                t2 = t2 + cm * vn[k]
                            r = 1.0 / (x + 0.5)
                            acc[k] = acc[k] + x * (t1 + r * t2)
                        if c <= 20:
                            vcur = vn
                    for k in range(GPS):
                        obuf[b, pl.ds(off + 16 * k, 16)] = acc[k]
                    return carry2

                lax.fori_loop(0, n_stripes, stripe, 0)
                return carry

            lax.fori_loop(0, CH_B, traj, 0)

        fire_chunk(0, 0)

        def body(i, carry):
            for j in range(2):
                ci = 2 * i + j
                nci = jnp.minimum(ci + 1, n_chunks - 1)
                fire_chunk(nci, 1 - j)
                wait_chunk(ci, j)
                compute_chunk(j)
                pltpu.sync_copy(
                    obuf, out_hbm.at[pl.ds(first + ci * CH_B, CH_B)]
                )
            return carry

        lax.fori_loop(0, n_chunks // 2, body, 0)
        wait_chunk(n_chunks - 1, 0)

    return run(ct, a_pad)


_B_SC = 1024


def kernel(candidates, a):
    B, T, _ = candidates.shape
    ct = jnp.transpose(candidates, (2, 0, 1))
    a_pad = jnp.zeros((_A_PAD,), jnp.float32).at[:_N_TERMS].set(a)
    sc_out = _sc_forward(ct, a_pad, _B_SC)
    tc_out = _tc_forward(ct, a_pad.reshape(1, _A_PAD), _B_SC, B - _B_SC)
    return jnp.concatenate([sc_out[:, :T], tc_out], axis=0)
